# baseline (speedup 1.0000x reference)
import functools

import jax
import jax.numpy as jnp
from jax import lax
from jax.experimental import pallas as pl
from jax.experimental.pallas import tpu as pltpu
from jax.experimental.pallas import tpu_sc as plsc

N_VEH = 6
N_PED = 2
LOGIT_DIM = 6

_B, _N, _D = 32, 256, 1024
_H = _D // 2
_T = _B * _N
_PAD = 16
_DUMP = _T + 8

_NSUB = 16
_NCORE = 2
_NW = _NSUB * _NCORE
_CHUNK = 64

_BLKB = 1024
_NSTEP_HEAD = _T // _BLKB
_MBLK = _T // (2 * _NSTEP_HEAD)

_SQRT_HALF = 0.7071067811865476


def _route_gather_body(x_hbm, t_hbm,
                       xs_hbm, idxv_hbm, idxp_hbm, cnt_hbm, cstage_hbm,
                       tv, lv, dv, dp, cw, call, ic, rows, sem):
    cid = lax.axis_index("c")
    sid = lax.axis_index("s")
    iota = lax.iota(jnp.int32, 16)
    tpw = _T // _NSUB
    base = sid * tpw

    pltpu.sync_copy(t_hbm.at[pl.ds(base, tpw)], tv)

    def cbody(j, carry):
        ov, op = carry
        t16 = tv[pl.ds(j * 16, 16)]
        lv[pl.ds(j * 16, 16)] = base + j * 16 + iota
        for k in range(16):
            ti = t16[k]
            ov = ov + jnp.where(ti == 0, 1, 0).astype(jnp.int32)
            op = op + jnp.where(ti == 1, 1, 0).astype(jnp.int32)
        return ov, op

    ov, op = lax.fori_loop(0, tpw // 16, cbody, (jnp.int32(0), jnp.int32(0)))

    cw[...] = jnp.where(iota == 0, ov, jnp.where(iota == 1, op, 0))
    pltpu.sync_copy(cw, cstage_hbm.at[pl.ds(sid * 16, 16)])
    plsc.subcore_barrier()
    pltpu.sync_copy(cstage_hbm, call)

    def pbody(wj, carry):
        offv, offp, cv, cp = carry
        r = call[pl.ds(wj * 16, 16)]
        a = r[0]
        b = r[1]
        sel = (wj < sid).astype(jnp.int32)
        return (offv + sel * a, offp + sel * b, cv + a, cp + b)

    offv, offp, cv, cp = lax.fori_loop(
        0, _NSUB, pbody,
        (jnp.int32(0), jnp.int32(0), jnp.int32(0), jnp.int32(0)))

    @pl.when(sid == 0)
    def _():
        cw[...] = jnp.where(iota == 0, cv, jnp.where(iota == 1, cp, 0))
        pltpu.sync_copy(cw, cnt_hbm)

    dump16 = jnp.full((16,), _DUMP, jnp.int32)

    def sbody(j, carry):
        rv, rp = carry
        t16 = tv[pl.ds(j * 16, 16)]
        dvv = dump16
        dpv = dump16
        for k in range(16):
            ti = t16[k]
            valv = jnp.where(ti == 0, offv + rv, _DUMP)
            valp = jnp.where(ti == 1, offp + rp, _DUMP)
            lane = iota == k
            dvv = jnp.where(lane, valv, dvv)
            dpv = jnp.where(lane, valp, dpv)
            rv = rv + jnp.where(ti == 0, 1, 0).astype(jnp.int32)
            rp = rp + jnp.where(ti == 1, 1, 0).astype(jnp.int32)
        dv[pl.ds(j * 16, 16)] = dvv
        dp[pl.ds(j * 16, 16)] = dpv
        return rv, rp

    lax.fori_loop(0, tpw // 16, sbody, (jnp.int32(0), jnp.int32(0)))
    pltpu.async_copy(lv, idxv_hbm.at[dv], sem).wait()
    pltpu.async_copy(lv, idxp_hbm.at[dp], sem).wait()
    plsc.subcore_barrier()

    w = sid * _NCORE + cid
    for head, cnt, idx_hbm in ((0, cv, idxv_hbm), (1, cp, idxp_hbm)):
        for c in range(256 // _CHUNK):
            start = w * 256 + c * _CHUNK

            @pl.when(start < cnt)
            def _(head=head, idx_hbm=idx_hbm, start=start):
                pltpu.sync_copy(idx_hbm.at[pl.ds(start, _CHUNK)], ic)
                for v in range(_CHUNK // 16):
                    val = ic[pl.ds(v * 16, 16)]
                    ic[pl.ds(v * 16, 16)] = jnp.minimum(
                        jnp.maximum(val, 0), _T - 1)
                pltpu.async_copy(x_hbm.at[ic], rows, sem).wait()
                pltpu.sync_copy(rows, xs_hbm.at[head, pl.ds(start, _CHUNK)])


def _route_gather(x, t):
    mesh = plsc.VectorSubcoreMesh(core_axis_name="c", subcore_axis_name="s")
    f = pl.kernel(
        _route_gather_body,
        out_type=[
            jax.ShapeDtypeStruct((2, _T, _D), jnp.float32),
            jax.ShapeDtypeStruct((_T + _PAD,), jnp.int32),
            jax.ShapeDtypeStruct((_T + _PAD,), jnp.int32),
            jax.ShapeDtypeStruct((16,), jnp.int32),
            jax.ShapeDtypeStruct((256,), jnp.int32),
        ],
        mesh=mesh,
        scratch_types=[
            pltpu.VMEM((_T // _NSUB,), jnp.int32),
            pltpu.VMEM((_T // _NSUB,), jnp.int32),
            pltpu.VMEM((_T // _NSUB,), jnp.int32),
            pltpu.VMEM((_T // _NSUB,), jnp.int32),
            pltpu.VMEM((16,), jnp.int32),
            pltpu.VMEM((256,), jnp.int32),
            pltpu.VMEM((_CHUNK,), jnp.int32),
            pltpu.VMEM((_CHUNK, _D), jnp.float32),
            pltpu.SemaphoreType.DMA,
        ],
    )
    return f(x, t)


def _mlp_body(cnt_ref, xs_ref, t_ref, w1v_ref, b1v_ref, w2v_ref, b2v_ref,
              w1p_ref, b1p_ref, w2p_ref, b2p_ref, ys_ref, mv_ref, mp_ref):
    i = pl.program_id(0)
    cv = cnt_ref[0]
    cp = cnt_ref[1]
    nbv = (cv + _BLKB - 1) // _BLKB
    nbp = (cp + _BLKB - 1) // _BLKB

    t = t_ref[...]
    mv_ref[...] = t == 0
    mp_ref[...] = t == 1

    def head(w1_ref, b1_ref, w2_ref, b2_ref):
        x = xs_ref[0]
        g = jnp.dot(x, w1_ref[...], preferred_element_type=jnp.float32)
        g = g + b1_ref[...]
        h = 0.5 * g * (1.0 + lax.erf(g * _SQRT_HALF))
        out = jnp.dot(h, w2_ref[...], preferred_element_type=jnp.float32)
        ys_ref[0] = jnp.transpose(out) + b2_ref[...]

    @pl.when((i < _NSTEP_HEAD) & (i < nbv))
    def _():
        head(w1v_ref, b1v_ref, w2v_ref, b2v_ref)

    @pl.when((i >= _NSTEP_HEAD) & (i - _NSTEP_HEAD < nbp))
    def _():
        head(w1p_ref, b1p_ref, w2p_ref, b2p_ref)


def _clamped_blk(i, c):
    nbv1 = jnp.maximum((c[0] + _BLKB - 1) // _BLKB - 1, 0)
    nbp1 = jnp.maximum((c[1] + _BLKB - 1) // _BLKB - 1, 0)
    return jnp.where(i < _NSTEP_HEAD,
                     jnp.minimum(i, nbv1),
                     jnp.minimum(jnp.maximum(i - _NSTEP_HEAD, 0), nbp1))


def _xs_map(i, c):
    return jnp.where(i < _NSTEP_HEAD, 0, 1), _clamped_blk(i, c), 0


def _ys_map(i, c):
    return jnp.where(i < _NSTEP_HEAD, 0, 1), 0, _clamped_blk(i, c)


def _expert_mlp(counts, xs, t2, W1v, b1v_r, W2v, b2v_r, W1p, b1p_r, w2p6, b2p_r):
    full = lambda i, c: (0, 0)
    grid_spec = pltpu.PrefetchScalarGridSpec(
        num_scalar_prefetch=1,
        grid=(2 * _NSTEP_HEAD,),
        in_specs=[
            pl.BlockSpec((1, _BLKB, _D), _xs_map),
            pl.BlockSpec((_MBLK, 1), lambda i, c: (i, 0)),
            pl.BlockSpec((_D, _H), full),
            pl.BlockSpec((1, _H), full),
            pl.BlockSpec((_H, LOGIT_DIM), full),
            pl.BlockSpec((LOGIT_DIM, 1), full),
            pl.BlockSpec((_D, _H), full),
            pl.BlockSpec((1, _H), full),
            pl.BlockSpec((_H, LOGIT_DIM), full),
            pl.BlockSpec((LOGIT_DIM, 1), full),
        ],
        out_specs=[
            pl.BlockSpec((1, LOGIT_DIM, _BLKB), _ys_map),
            pl.BlockSpec((_MBLK, 1), lambda i, c: (i, 0)),
            pl.BlockSpec((_MBLK, 1), lambda i, c: (i, 0)),
        ],
    )
    return pl.pallas_call(
        _mlp_body,
        grid_spec=grid_spec,
        out_shape=[
            jax.ShapeDtypeStruct((2, LOGIT_DIM, _T), jnp.float32),
            jax.ShapeDtypeStruct((_T, 1), jnp.bool_),
            jax.ShapeDtypeStruct((_T, 1), jnp.bool_),
        ],
    )(counts, xs, t2, W1v, b1v_r, W2v, b2v_r, W1p, b1p_r, w2p6, b2p_r)


def _scatter_body(ys_hbm, idxv_hbm, idxp_hbm, cnt_hbm, t_hbm, z_hbm,
                  outb_hbm, zv, tc, ic, dref, eref, yb, cvec, sem):
    cid = lax.axis_index("c")
    sid = lax.axis_index("s")
    w = sid * _NCORE + cid
    iota = lax.iota(jnp.int32, 16)

    pltpu.sync_copy(z_hbm, zv)
    pltpu.sync_copy(cnt_hbm, cvec)
    cval = cvec[...]
    cv = cval[0]
    cp = cval[1]

    def col_scatter(col):
        for v in range(_CHUNK // 16):
            dval = dref[pl.ds(v * 16, 16)]
            eref[pl.ds(v * 16, 16)] = dval * LOGIT_DIM + col

    for c in range(256 // _CHUNK):
        start = w * 256 + c * _CHUNK
        pltpu.sync_copy(t_hbm.at[pl.ds(start, _CHUNK)], tc)
        for v in range(_CHUNK // 16):
            t16 = tc[pl.ds(v * 16, 16)]
            pos = start + v * 16 + iota
            ri = jnp.where(t16 == 0, 1, 0) + jnp.where(t16 == 1, 1, 0)
            dref[pl.ds(v * 16, 16)] = jnp.where(ri > 0, _DUMP, pos)
        for col in range(LOGIT_DIM):
            col_scatter(col)
            pltpu.async_copy(zv, outb_hbm.at[eref], sem).wait()

    for head, cnt, idx_hbm in ((0, cv, idxv_hbm), (1, cp, idxp_hbm)):
        for c in range(256 // _CHUNK):
            start = w * 256 + c * _CHUNK

            @pl.when(start < cnt)
            def _(head=head, cnt=cnt, idx_hbm=idx_hbm, start=start):
                pltpu.sync_copy(idx_hbm.at[pl.ds(start, _CHUNK)], ic)
                for v in range(_CHUNK // 16):
                    pos = start + v * 16 + iota
                    val = ic[pl.ds(v * 16, 16)]
                    val = jnp.minimum(jnp.maximum(val, 0), _T - 1)
                    dref[pl.ds(v * 16, 16)] = jnp.where(pos < cnt, val, _DUMP)
                for col in range(LOGIT_DIM):
                    col_scatter(col)
                    pltpu.sync_copy(
                        ys_hbm.at[pl.ds((head * LOGIT_DIM + col) * _T + start,
                                        _CHUNK)], yb)
                    pltpu.async_copy(yb, outb_hbm.at[eref], sem).wait()


def _scatter(ys_flat, idxv, idxp, counts, t, z):
    mesh = plsc.VectorSubcoreMesh(core_axis_name="c", subcore_axis_name="s")
    f = pl.kernel(
        _scatter_body,
        out_type=jax.ShapeDtypeStruct(((_T + _PAD) * LOGIT_DIM,), jnp.float32),
        mesh=mesh,
        scratch_types=[
            pltpu.VMEM((_CHUNK,), jnp.float32),
            pltpu.VMEM((_CHUNK,), jnp.int32),
            pltpu.VMEM((_CHUNK,), jnp.int32),
            pltpu.VMEM((_CHUNK,), jnp.int32),
            pltpu.VMEM((_CHUNK,), jnp.int32),
            pltpu.VMEM((_CHUNK,), jnp.float32),
            pltpu.VMEM((16,), jnp.int32),
            pltpu.SemaphoreType.DMA,
        ],
    )
    return f(ys_flat, idxv, idxp, counts, t, z)


def kernel(repr3, agent_type_ids, W1v, b1v, W2v, b2v, W1p, b1p, W2p, b2p):
    x = repr3.reshape(_T, _D)
    t = agent_type_ids.reshape(_T)
    t2 = agent_type_ids.reshape(_T, 1)

    w2p6 = jnp.pad(W2p, ((0, 0), (0, LOGIT_DIM - N_PED)))
    b1v_r = b1v.reshape(1, _H)
    b1p_r = b1p.reshape(1, _H)
    b2v_r = b2v.reshape(LOGIT_DIM, 1)
    b2p_r = jnp.pad(b2p, (0, LOGIT_DIM - N_PED)).reshape(LOGIT_DIM, 1)
    z = jnp.zeros((_CHUNK,), jnp.float32)

    xs, idxv, idxp, counts, _ = _route_gather(x, t)
    ys, mv, mp = _expert_mlp(counts, xs, t2, W1v, b1v_r, W2v, b2v_r,
                             W1p, b1p_r, w2p6, b2p_r)
    outb = _scatter(ys.reshape(2 * LOGIT_DIM * _T), idxv, idxp, counts, t, z)

    return (outb[:_T * LOGIT_DIM].reshape(_B, _N, LOGIT_DIM),
            mv.reshape(_B, _N),
            mp.reshape(_B, _N))


# speedup vs baseline: 123.5106x; 123.5106x over previous
import jax
import jax.numpy as jnp
from jax.experimental import pallas as pl

N_VEH = 6
N_PED = 2
LOGIT_DIM = 6

_B, _N, _D = 32, 256, 1024
_H = _D // 2
_T = _B * _N
_BLK = 1024

_SQRT_HALF = 0.7071067811865476


def _body(x_ref, t_ref, w1v_ref, b1v_ref, w2v_ref, b2v_ref,
          w1p_ref, b1p_ref, w2p_ref, b2p_ref,
          out_ref, mv_ref, mp_ref):
    x = x_ref[...]
    t = t_ref[...]
    mv = t == 0
    mp = t == 1

    gv = jnp.dot(x, w1v_ref[...], preferred_element_type=jnp.float32) + b1v_ref[...]
    hv = 0.5 * gv * (1.0 + jax.lax.erf(gv * _SQRT_HALF))
    gp = jnp.dot(x, w1p_ref[...], preferred_element_type=jnp.float32) + b1p_ref[...]
    hp = 0.5 * gp * (1.0 + jax.lax.erf(gp * _SQRT_HALF))

    hv = hv * mv.astype(jnp.float32)
    hp = hp * mp.astype(jnp.float32)
    out = (jnp.dot(hv, w2v_ref[...], preferred_element_type=jnp.float32)
           + jnp.dot(hp, w2p_ref[...], preferred_element_type=jnp.float32))
    out = out + jnp.where(mv, b2v_ref[...], 0.0) + jnp.where(mp, b2p_ref[...], 0.0)
    out_ref[...] = out
    mv_ref[...] = mv
    mp_ref[...] = mp


def kernel(repr3, agent_type_ids, W1v, b1v, W2v, b2v, W1p, b1p, W2p, b2p):
    x = repr3.reshape(_T, _D)
    t = agent_type_ids.reshape(_T, 1)

    w2p6 = jnp.pad(W2p, ((0, 0), (0, LOGIT_DIM - N_PED)))
    b1v_r = b1v.reshape(1, _H)
    b1p_r = b1p.reshape(1, _H)
    b2v_r = b2v.reshape(1, LOGIT_DIM)
    b2p_r = jnp.pad(b2p, (0, LOGIT_DIM - N_PED)).reshape(1, LOGIT_DIM)

    nblk = _T // _BLK
    full = lambda i: (0, 0)
    out, mv, mp = pl.pallas_call(
        _body,
        grid=(nblk,),
        in_specs=[
            pl.BlockSpec((_BLK, _D), lambda i: (i, 0)),
            pl.BlockSpec((_BLK, 1), lambda i: (i, 0)),
            pl.BlockSpec((_D, _H), full),
            pl.BlockSpec((1, _H), full),
            pl.BlockSpec((_H, LOGIT_DIM), full),
            pl.BlockSpec((1, LOGIT_DIM), full),
            pl.BlockSpec((_D, _H), full),
            pl.BlockSpec((1, _H), full),
            pl.BlockSpec((_H, LOGIT_DIM), full),
            pl.BlockSpec((1, LOGIT_DIM), full),
        ],
        out_specs=[
            pl.BlockSpec((_BLK, LOGIT_DIM), lambda i: (i, 0)),
            pl.BlockSpec((_BLK, 1), lambda i: (i, 0)),
            pl.BlockSpec((_BLK, 1), lambda i: (i, 0)),
        ],
        out_shape=[
            jax.ShapeDtypeStruct((_T, LOGIT_DIM), jnp.float32),
            jax.ShapeDtypeStruct((_T, 1), jnp.bool_),
            jax.ShapeDtypeStruct((_T, 1), jnp.bool_),
        ],
    )(x, t, W1v, b1v_r, W2v, b2v_r, W1p, b1p_r, w2p6, b2p_r)

    return (out.reshape(_B, _N, LOGIT_DIM),
            mv.reshape(_B, _N),
            mp.reshape(_B, _N))
